# baseline (speedup 1.0000x reference)
import jax
import jax.numpy as jnp
from jax import lax
from jax.experimental import pallas as pl
from jax.experimental.pallas import tpu as pltpu
from jax.experimental.pallas import tpu_sc as plsc

_NUM_CORES = 2
_NUM_SUBCORES = 16
_NW = _NUM_CORES * _NUM_SUBCORES
_CHUNK = 8
_NB = 3


def _embed_body(table_hbm, ids_hbm, out_hbm, idx_v, bufs, gsems, osems):
    batch, seq = ids_hbm.shape
    b_per_w = (batch * seq) // _NW
    n_chunks = b_per_w // _CHUNK
    w_per_b = seq // b_per_w

    wid = lax.axis_index("s") * _NUM_CORES + lax.axis_index("c")
    brow = wid // w_per_b
    col0 = (wid % w_per_b) * b_per_w

    pltpu.sync_copy(ids_hbm.at[brow, pl.ds(col0, b_per_w)], idx_v)

    def start_gather(c, slot):
        pltpu.async_copy(
            table_hbm.at[idx_v.at[pl.ds(c * _CHUNK, _CHUNK)]],
            bufs[slot],
            gsems[slot],
        )

    def wait_gather(slot):
        pltpu.make_async_copy(
            table_hbm.at[idx_v.at[pl.ds(0, _CHUNK)]], bufs[slot], gsems[slot]
        ).wait()

    def start_out(c, slot):
        pltpu.async_copy(
            bufs[slot],
            out_hbm.at[brow, pl.ds(col0 + c * _CHUNK, _CHUNK)],
            osems[slot],
        )

    def wait_out(c, slot):
        pltpu.make_async_copy(
            bufs[slot],
            out_hbm.at[brow, pl.ds(col0 + c * _CHUNK, _CHUNK)],
            osems[slot],
        ).wait()

    def body(c, slot, pslot, has_wo, has_g):
        wait_gather(slot)
        start_out(c, slot)
        if has_wo:
            wait_out(c - 1, pslot)
        if has_g:
            start_gather(c + 2, pslot)

    start_gather(0, 0)
    start_gather(1, 1)
    body(0, 0, 2, False, True)
    body(1, 1, 0, True, True)

    @pl.loop(2, n_chunks - 3, step=_NB)
    def _(g0):
        for b in range(_NB):
            body(g0 + b, (2 + b) % _NB, (1 + b) % _NB, True, True)

    body(n_chunks - 3, (n_chunks - 3) % _NB, (n_chunks - 4) % _NB, True, True)
    body(n_chunks - 2, (n_chunks - 2) % _NB, (n_chunks - 3) % _NB, True, False)
    body(n_chunks - 1, (n_chunks - 1) % _NB, (n_chunks - 2) % _NB, True, False)
    wait_out(n_chunks - 1, (n_chunks - 1) % _NB)


def kernel(input_ids, table):
    batch, seq = input_ids.shape
    vocab, d = table.shape
    ids = input_ids.astype(jnp.int32)

    mesh = plsc.VectorSubcoreMesh(
        core_axis_name="c",
        subcore_axis_name="s",
        num_cores=_NUM_CORES,
        num_subcores=_NUM_SUBCORES,
    )

    run = pl.kernel(
        _embed_body,
        out_type=jax.ShapeDtypeStruct((batch, seq, d), jnp.float32),
        mesh=mesh,
        scratch_types=[
            pltpu.VMEM(((batch * seq) // _NW,), jnp.int32),
            [pltpu.VMEM((_CHUNK, d), jnp.float32) for _ in range(_NB)],
            [pltpu.SemaphoreType.DMA for _ in range(_NB)],
            [pltpu.SemaphoreType.DMA for _ in range(_NB)],
        ],
    )
    return run(table, ids)


# speedup vs baseline: 1.0099x; 1.0099x over previous
import jax
import jax.numpy as jnp
from jax import lax
from jax.experimental import pallas as pl
from jax.experimental.pallas import tpu as pltpu
from jax.experimental.pallas import tpu_sc as plsc

_NUM_CORES = 2
_NUM_SUBCORES = 16
_NW = _NUM_CORES * _NUM_SUBCORES
_CHUNK = 8
_NB = 2


def _embed_body(table_hbm, ids_hbm, out_hbm, idx_v, bufs, gsems, osems):
    batch, seq = ids_hbm.shape
    b_per_w = (batch * seq) // _NW
    n_chunks = b_per_w // _CHUNK
    w_per_b = seq // b_per_w

    wid = lax.axis_index("s") * _NUM_CORES + lax.axis_index("c")
    brow = wid // w_per_b
    col0 = (wid % w_per_b) * b_per_w

    pltpu.sync_copy(ids_hbm.at[brow, pl.ds(col0, b_per_w)], idx_v)

    def start_gather(c, slot):
        pltpu.async_copy(
            table_hbm.at[idx_v.at[pl.ds(c * _CHUNK, _CHUNK)]],
            bufs[slot],
            gsems[slot],
        )

    def wait_gather(slot):
        pltpu.make_async_copy(
            table_hbm.at[idx_v.at[pl.ds(0, _CHUNK)]], bufs[slot], gsems[slot]
        ).wait()

    def start_out(c, slot):
        pltpu.async_copy(
            bufs[slot],
            out_hbm.at[brow, pl.ds(col0 + c * _CHUNK, _CHUNK)],
            osems[slot],
        )

    def wait_out(c, slot):
        pltpu.make_async_copy(
            bufs[slot],
            out_hbm.at[brow, pl.ds(col0 + c * _CHUNK, _CHUNK)],
            osems[slot],
        ).wait()

    def drain(c, slot):
        wait_gather(slot)
        start_out(c, slot)
        wait_out(c, slot)

    start_gather(0, 0)
    start_gather(1, 1)

    @pl.loop(0, n_chunks - 2, step=2)
    def _(g0):
        for b in range(2):
            c = g0 + b
            drain(c, b)
            start_gather(c + 2, b)

    drain(n_chunks - 2, 0)
    drain(n_chunks - 1, 1)


def kernel(input_ids, table):
    batch, seq = input_ids.shape
    vocab, d = table.shape
    ids = input_ids.astype(jnp.int32)

    mesh = plsc.VectorSubcoreMesh(
        core_axis_name="c",
        subcore_axis_name="s",
        num_cores=_NUM_CORES,
        num_subcores=_NUM_SUBCORES,
    )

    run = pl.kernel(
        _embed_body,
        out_type=jax.ShapeDtypeStruct((batch, seq, d), jnp.float32),
        mesh=mesh,
        scratch_types=[
            pltpu.VMEM(((batch * seq) // _NW,), jnp.int32),
            [pltpu.VMEM((_CHUNK, d), jnp.float32) for _ in range(_NB)],
            [pltpu.SemaphoreType.DMA for _ in range(_NB)],
            [pltpu.SemaphoreType.DMA for _ in range(_NB)],
        ],
    )
    return run(table, ids)
